# baseline (speedup 1.0000x reference)
import functools

import jax
import jax.numpy as jnp
import numpy as np
from jax import lax
from jax.experimental import pallas as pl
from jax.experimental.pallas import tpu as pltpu
from jax.experimental.pallas import tpu_sc as plsc

N = 10000
K = 32
CIN = 128
CMID = 32
COUT = 128
KP = 15
G = 8
SIGMA = 0.75
TW = 48

_NW = 32
_CH = 80
_CPW = 125
_RPW = _CH * _CPW
_NPAD = _NW * _RPW

_EPS = 1e-5


def _leaky(x):
    return jnp.where(x >= 0, x, 0.1 * x)


def _gn(x, gamma, beta, mask, n_per_group):
    s = jnp.sum(x, axis=0, keepdims=True)
    ss = jnp.sum(x * x, axis=0, keepdims=True)
    sg = jnp.dot(s, mask, preferred_element_type=jnp.float32)
    ssg = jnp.dot(ss, mask, preferred_element_type=jnp.float32)
    mean = sg / n_per_group
    var = ssg / n_per_group - mean * mean
    inv = lax.rsqrt(var + _EPS)
    return (x - mean) * inv * gamma + beta


def _k1_body(s_feats_ref, s_points_ref, w1_ref, b1_ref, g1_ref, be1_ref,
             m32_ref, out_ref):
    x = jnp.dot(s_feats_ref[...], w1_ref[...], preferred_element_type=jnp.float32)
    x = x + b1_ref[...]
    x = _gn(x, g1_ref[...], be1_ref[...], m32_ref[...], float((CMID // G) * N))
    x = _leaky(x)
    pad = jnp.zeros((N, TW - CMID - 3), jnp.float32)
    out_ref[...] = jnp.concatenate([x, s_points_ref[...], pad], axis=1)


def _sc_gather(table, idx2d):
    mesh = plsc.VectorSubcoreMesh(core_axis_name="c", subcore_axis_name="s")

    @functools.partial(
        pl.kernel,
        mesh=mesh,
        compiler_params=pltpu.CompilerParams(use_tc_tiling_on_sc=False),
        out_type=jax.ShapeDtypeStruct((K // 2, N, 128), jnp.float32),
        scratch_types=[
            pltpu.VMEM((_CPW, _CH), jnp.int32),
            pltpu.VMEM((_CH, TW), jnp.float32),
            pltpu.SemaphoreType.DMA,
        ],
    )
    def gather_kernel(table_hbm, idx_hbm, out_hbm, idx_v, rows_v, sem):
        s_idx = lax.axis_index("s")
        c_idx = lax.axis_index("c")
        wid = s_idx * 2 + c_idx
        pltpu.sync_copy(idx_hbm.at[wid], idx_v)

        def body(i, carry):
            pltpu.async_copy(table_hbm.at[idx_v.at[i]], rows_v, sem).wait()

            @pl.when(c_idx == 0)
            def _():
                pltpu.sync_copy(
                    rows_v,
                    out_hbm.at[s_idx, pl.ds(i * _CH, _CH), pl.ds(0, TW)])

            @pl.when(c_idx == 1)
            def _():
                pltpu.sync_copy(
                    rows_v,
                    out_hbm.at[s_idx, pl.ds(i * _CH, _CH), pl.ds(TW, TW)])

            return carry

        lax.fori_loop(0, _CPW, body, 0)

    return gather_kernel(table, idx2d)


def _k2_body(g_ref, q_ref, kpt_ref, e_ref, t_ref, wkp_ref, bias_ref, out_ref,
             *, bm):
    qx = q_ref[:, 0:1]
    qy = q_ref[:, 1:2]
    qz = q_ref[:, 2:3]
    kpx = kpt_ref[0:1, :]
    kpy = kpt_ref[1:2, :]
    kpz = kpt_ref[2:3, :]
    acc = jnp.zeros((bm, KP * CMID), jnp.float32)
    cnt = jnp.zeros((bm, 1), jnp.float32)
    for k in range(K):
        sl, off = k // 2, (k % 2) * TW
        feats = g_ref[sl, :, off:off + CMID]
        dx = g_ref[sl, :, off + CMID:off + CMID + 1] - qx
        dy = g_ref[sl, :, off + CMID + 1:off + CMID + 2] - qy
        dz = g_ref[sl, :, off + CMID + 2:off + CMID + 3] - qz
        ddx = dx - kpx
        ddy = dy - kpy
        ddz = dz - kpz
        sq_d = ddx * ddx + ddy * ddy + ddz * ddz
        nw = jnp.maximum(1.0 - jnp.sqrt(sq_d) * (1.0 / SIGMA), 0.0)
        nw_full = jnp.dot(nw, e_ref[...], preferred_element_type=jnp.float32)
        f_full = jnp.dot(feats, t_ref[...], preferred_element_type=jnp.float32)
        acc = acc + nw_full * f_full
        nfs = jnp.sum(feats, axis=1, keepdims=True)
        cnt = cnt + jnp.where(nfs > 0.0, 1.0, 0.0)
    nnum = jnp.maximum(cnt, 1.0)
    out = jnp.dot(acc, wkp_ref[...], preferred_element_type=jnp.float32)
    out_ref[...] = out / nnum + bias_ref[...]


def _k3_body(xm_ref, s_feats_ref, gc_ref, bc_ref, w2_ref, b2_ref, g2_ref,
             be2_ref, m32_ref, m128_ref, out_ref):
    x = _gn(xm_ref[...], gc_ref[...], bc_ref[...], m32_ref[...],
            float((CMID // G) * N))
    x = _leaky(x)
    x = jnp.dot(x, w2_ref[...], preferred_element_type=jnp.float32)
    x = x + b2_ref[...]
    x = _gn(x, g2_ref[...], be2_ref[...], m128_ref[...], float((COUT // G) * N))
    x = x + s_feats_ref[...]
    out_ref[...] = _leaky(x)


def kernel(s_feats, q_points, s_points, neighbor_indices, W1, b1, g1, be1,
           kernel_points, kp_weights, kp_bias, gc, bc, W2, b2, g2, be2):
    m32 = jnp.asarray(np.kron(np.eye(G), np.ones((CMID // G, CMID // G))),
                      jnp.float32)
    m128 = jnp.asarray(np.kron(np.eye(G), np.ones((COUT // G, COUT // G))),
                       jnp.float32)
    e_np = np.zeros((KP, KP * CMID), np.float32)
    for p in range(KP):
        e_np[p, p * CMID:(p + 1) * CMID] = 1.0
    t_np = np.zeros((CMID, KP * CMID), np.float32)
    for p in range(KP):
        t_np[:, p * CMID:(p + 1) * CMID] = np.eye(CMID, dtype=np.float32)
    e_c = jnp.asarray(e_np)
    t_c = jnp.asarray(t_np)
    wkp = kp_weights.reshape(KP * CMID, CMID)

    table = pl.pallas_call(
        _k1_body,
        out_shape=jax.ShapeDtypeStruct((N, TW), jnp.float32),
    )(s_feats, s_points, W1, b1.reshape(1, CMID), g1.reshape(1, CMID),
      be1.reshape(1, CMID), m32)

    idx = neighbor_indices.astype(jnp.int32).T.reshape(_NW, _CPW, _CH)
    g3 = _sc_gather(table, idx)

    bm = 400
    nblk = N // bm
    out_mid = pl.pallas_call(
        functools.partial(_k2_body, bm=bm),
        grid=(nblk,),
        in_specs=[
            pl.BlockSpec((K // 2, bm, 128), lambda i: (0, i, 0)),
            pl.BlockSpec((bm, 3), lambda i: (i, 0)),
            pl.BlockSpec((3, KP), lambda i: (0, 0)),
            pl.BlockSpec((KP, KP * CMID), lambda i: (0, 0)),
            pl.BlockSpec((CMID, KP * CMID), lambda i: (0, 0)),
            pl.BlockSpec((KP * CMID, CMID), lambda i: (0, 0)),
            pl.BlockSpec((1, CMID), lambda i: (0, 0)),
        ],
        out_specs=pl.BlockSpec((bm, CMID), lambda i: (i, 0)),
        out_shape=jax.ShapeDtypeStruct((N, CMID), jnp.float32),
    )(g3, q_points, kernel_points.T, e_c, t_c, wkp,
      kp_bias.reshape(1, CMID))

    out = pl.pallas_call(
        _k3_body,
        out_shape=jax.ShapeDtypeStruct((N, COUT), jnp.float32),
    )(out_mid, s_feats, gc.reshape(1, CMID), bc.reshape(1, CMID), W2,
      b2.reshape(1, COUT), g2.reshape(1, COUT), be2.reshape(1, COUT), m32,
      m128)
    return out[:, None, :]


# speedup vs baseline: 1.1048x; 1.1048x over previous
import functools

import jax
import jax.numpy as jnp
import numpy as np
from jax import lax
from jax.experimental import pallas as pl
from jax.experimental.pallas import tpu as pltpu
from jax.experimental.pallas import tpu_sc as plsc

N = 10000
K = 32
CIN = 128
CMID = 32
COUT = 128
KP = 15
G = 8
SIGMA = 0.75
TW = 48

_NW = 32
_CH = 80
_CPW = 125
_RPW = _CH * _CPW
_NPAD = _NW * _RPW

_EPS = 1e-5


def _leaky(x):
    return jnp.where(x >= 0, x, 0.1 * x)


def _gn(x, gamma, beta, mask, n_per_group):
    s = jnp.sum(x, axis=0, keepdims=True)
    ss = jnp.sum(x * x, axis=0, keepdims=True)
    sg = jnp.dot(s, mask, preferred_element_type=jnp.float32)
    ssg = jnp.dot(ss, mask, preferred_element_type=jnp.float32)
    mean = sg / n_per_group
    var = ssg / n_per_group - mean * mean
    inv = lax.rsqrt(var + _EPS)
    return (x - mean) * inv * gamma + beta


def _k1_body(s_feats_ref, s_points_ref, w1_ref, b1_ref, g1_ref, be1_ref,
             m32_ref, out_ref):
    x = jnp.dot(s_feats_ref[...], w1_ref[...], preferred_element_type=jnp.float32)
    x = x + b1_ref[...]
    x = _gn(x, g1_ref[...], be1_ref[...], m32_ref[...], float((CMID // G) * N))
    x = _leaky(x)
    pad = jnp.zeros((N, TW - CMID - 3), jnp.float32)
    out_ref[...] = jnp.concatenate([x, s_points_ref[...], pad], axis=1)


def _sc_gather(table, idx2d):
    mesh = plsc.VectorSubcoreMesh(core_axis_name="c", subcore_axis_name="s")

    @functools.partial(
        pl.kernel,
        mesh=mesh,
        compiler_params=pltpu.CompilerParams(use_tc_tiling_on_sc=False),
        out_type=jax.ShapeDtypeStruct((K, N, 128), jnp.float32),
        scratch_types=[
            pltpu.VMEM((_CPW, _CH), jnp.int32),
            pltpu.VMEM((_CH, TW), jnp.float32),
            pltpu.SemaphoreType.DMA,
        ],
    )
    def gather_kernel(table_hbm, idx_hbm, out_hbm, idx_v, rows_v, sem):
        wid = lax.axis_index("s") * 2 + lax.axis_index("c")
        pltpu.sync_copy(idx_hbm.at[wid], idx_v)

        def body(i, carry):
            pltpu.async_copy(table_hbm.at[idx_v.at[i]], rows_v, sem).wait()
            pltpu.sync_copy(
                rows_v, out_hbm.at[wid, pl.ds(i * _CH, _CH), pl.ds(0, TW)])
            return carry

        lax.fori_loop(0, _CPW, body, 0)

    return gather_kernel(table, idx2d)


def _k2_body(g_ref, q_ref, kpt_ref, e_ref, t_ref, wkp_ref, bias_ref, out_ref,
             *, bm):
    qx = q_ref[:, 0:1]
    qy = q_ref[:, 1:2]
    qz = q_ref[:, 2:3]
    kpx = kpt_ref[0:1, :]
    kpy = kpt_ref[1:2, :]
    kpz = kpt_ref[2:3, :]
    acc = jnp.zeros((bm, KP * CMID), jnp.float32)
    cnt = jnp.zeros((bm, 1), jnp.float32)
    for k in range(K):
        feats = g_ref[k, :, 0:CMID]
        dx = g_ref[k, :, CMID:CMID + 1] - qx
        dy = g_ref[k, :, CMID + 1:CMID + 2] - qy
        dz = g_ref[k, :, CMID + 2:CMID + 3] - qz
        ddx = dx - kpx
        ddy = dy - kpy
        ddz = dz - kpz
        sq_d = ddx * ddx + ddy * ddy + ddz * ddz
        nw = jnp.maximum(1.0 - jnp.sqrt(sq_d) * (1.0 / SIGMA), 0.0)
        nw_full = jnp.dot(nw, e_ref[...], preferred_element_type=jnp.float32)
        f_full = jnp.dot(feats, t_ref[...], preferred_element_type=jnp.float32)
        acc = acc + nw_full * f_full
        nfs = jnp.sum(feats, axis=1, keepdims=True)
        cnt = cnt + jnp.where(nfs > 0.0, 1.0, 0.0)
    nnum = jnp.maximum(cnt, 1.0)
    out = jnp.dot(acc, wkp_ref[...], preferred_element_type=jnp.float32)
    out_ref[...] = out / nnum + bias_ref[...]


def _k3_body(xm_ref, s_feats_ref, gc_ref, bc_ref, w2_ref, b2_ref, g2_ref,
             be2_ref, m32_ref, m128_ref, out_ref):
    x = _gn(xm_ref[...], gc_ref[...], bc_ref[...], m32_ref[...],
            float((CMID // G) * N))
    x = _leaky(x)
    x = jnp.dot(x, w2_ref[...], preferred_element_type=jnp.float32)
    x = x + b2_ref[...]
    x = _gn(x, g2_ref[...], be2_ref[...], m128_ref[...], float((COUT // G) * N))
    x = x + s_feats_ref[...]
    out_ref[...] = _leaky(x)


def kernel(s_feats, q_points, s_points, neighbor_indices, W1, b1, g1, be1,
           kernel_points, kp_weights, kp_bias, gc, bc, W2, b2, g2, be2):
    m32 = jnp.asarray(np.kron(np.eye(G), np.ones((CMID // G, CMID // G))),
                      jnp.float32)
    m128 = jnp.asarray(np.kron(np.eye(G), np.ones((COUT // G, COUT // G))),
                       jnp.float32)
    e_np = np.zeros((KP, KP * CMID), np.float32)
    for p in range(KP):
        e_np[p, p * CMID:(p + 1) * CMID] = 1.0
    t_np = np.zeros((CMID, KP * CMID), np.float32)
    for p in range(KP):
        t_np[:, p * CMID:(p + 1) * CMID] = np.eye(CMID, dtype=np.float32)
    e_c = jnp.asarray(e_np)
    t_c = jnp.asarray(t_np)
    wkp = kp_weights.reshape(KP * CMID, CMID)

    table = pl.pallas_call(
        _k1_body,
        out_shape=jax.ShapeDtypeStruct((N, TW), jnp.float32),
    )(s_feats, s_points, W1, b1.reshape(1, CMID), g1.reshape(1, CMID),
      be1.reshape(1, CMID), m32)

    idx = neighbor_indices.astype(jnp.int32).T.reshape(_NW, _CPW, _CH)
    g3 = _sc_gather(table, idx)

    bm = 400
    nblk = N // bm
    out_mid = pl.pallas_call(
        functools.partial(_k2_body, bm=bm),
        grid=(nblk,),
        in_specs=[
            pl.BlockSpec((K, bm, 128), lambda i: (0, i, 0)),
            pl.BlockSpec((bm, 3), lambda i: (i, 0)),
            pl.BlockSpec((3, KP), lambda i: (0, 0)),
            pl.BlockSpec((KP, KP * CMID), lambda i: (0, 0)),
            pl.BlockSpec((CMID, KP * CMID), lambda i: (0, 0)),
            pl.BlockSpec((KP * CMID, CMID), lambda i: (0, 0)),
            pl.BlockSpec((1, CMID), lambda i: (0, 0)),
        ],
        out_specs=pl.BlockSpec((bm, CMID), lambda i: (i, 0)),
        out_shape=jax.ShapeDtypeStruct((N, CMID), jnp.float32),
    )(g3, q_points, kernel_points.T, e_c, t_c, wkp,
      kp_bias.reshape(1, CMID))

    out = pl.pallas_call(
        _k3_body,
        out_shape=jax.ShapeDtypeStruct((N, COUT), jnp.float32),
    )(out_mid, s_feats, gc.reshape(1, CMID), bc.reshape(1, CMID), W2,
      b2.reshape(1, COUT), g2.reshape(1, COUT), be2.reshape(1, COUT), m32,
      m128)
    return out[:, None, :]
